# Initial kernel scaffold; baseline (speedup 1.0000x reference)
import jax
import jax.numpy as jnp
from jax.experimental import pallas as pl


def kernel(nids, ts, left_weight, right_weight, prev_ts_table):
    raise NotImplementedError("write your pallas kernel here")


# speedup vs baseline: 1.6239x; 1.6239x over previous
import functools

import jax
import jax.numpy as jnp
from jax import lax
from jax.experimental import pallas as pl
from jax.experimental.pallas import tpu as pltpu
from jax.experimental.pallas import tpu_sc as plsc


@functools.lru_cache(maxsize=None)
def _build(B, D, N, NC, NS):
    NW = NC * NS
    b_per_w = B // NW
    C = 128
    NCH = b_per_w // C

    mesh = plsc.VectorSubcoreMesh(core_axis_name="c", subcore_axis_name="s")

    @functools.partial(
        pl.kernel,
        mesh=mesh,
        out_type=(
            jax.ShapeDtypeStruct((B, D), jnp.float32),
            jax.ShapeDtypeStruct((B, D), jnp.float32),
            jax.ShapeDtypeStruct((B,), jnp.float32),
        ),
        scratch_types=[
            pltpu.VMEM((NCH, C), jnp.int32),
            pltpu.VMEM((NCH, C), jnp.float32),
            pltpu.VMEM((C, D), jnp.float32),
            pltpu.VMEM((C, D), jnp.float32),
            pltpu.VMEM((C,), jnp.float32),
            pltpu.SemaphoreType.DMA,
            pltpu.SemaphoreType.DMA,
            pltpu.SemaphoreType.DMA,
        ],
    )
    def k(nids_hbm, ts_hbm, left_hbm, right_hbm, pts_hbm,
          hl_out, hr_out, pts_out,
          idx_v, ts_v, lrows_v, rrows_v, pts_v,
          lsem, rsem, psem):
        wid = lax.axis_index("s") * NC + lax.axis_index("c")
        base = wid * b_per_w
        pltpu.sync_copy(nids_hbm.at[wid], idx_v)
        pltpu.sync_copy(ts_hbm.at[wid], ts_v)
        for j in range(NCH):
            ij = idx_v.at[j]
            lcp = pltpu.async_copy(left_hbm.at[ij], lrows_v, lsem)
            rcp = pltpu.async_copy(right_hbm.at[ij], rrows_v, rsem)
            pcp = pltpu.async_copy(pts_hbm.at[ij], pts_v, psem)
            pcp.wait()
            for i in range(C // 16):
                sl = pl.ds(i * 16, 16)
                pts_v[sl] = jnp.minimum(pts_v[sl], ts_v[j, sl])
            pltpu.sync_copy(pts_v, pts_out.at[pl.ds(base + j * C, C)])
            lcp.wait()
            pltpu.sync_copy(lrows_v, hl_out.at[pl.ds(base + j * C, C)])
            rcp.wait()
            pltpu.sync_copy(rrows_v, hr_out.at[pl.ds(base + j * C, C)])

    return k, NW, NCH, C


def kernel(nids, ts, left_weight, right_weight, prev_ts_table):
    B, = nids.shape
    N, D = left_weight.shape
    info = plsc.get_sparse_core_info()
    k, NW, NCH, C = _build(B, D, N, info.num_cores, info.num_subcores)
    nids3 = nids.astype(jnp.int32).reshape(NW, NCH, C)
    ts3 = ts.reshape(NW, NCH, C)
    h_left, h_right, prev_ts = k(nids3, ts3, left_weight, right_weight,
                                 prev_ts_table)
    return (h_left, h_right, prev_ts)


# speedup vs baseline: 1.6818x; 1.0357x over previous
import functools

import jax
import jax.numpy as jnp
from jax import lax
from jax.experimental import pallas as pl
from jax.experimental.pallas import tpu as pltpu
from jax.experimental.pallas import tpu_sc as plsc


@functools.lru_cache(maxsize=None)
def _build(B, D, N, NC, NS):
    NW = NC * NS
    b_per_w = B // NW
    C = 128
    NCH = b_per_w // C

    mesh = plsc.VectorSubcoreMesh(core_axis_name="c", subcore_axis_name="s")

    @functools.partial(
        pl.kernel,
        mesh=mesh,
        out_type=(
            jax.ShapeDtypeStruct((B, D), jnp.float32),
            jax.ShapeDtypeStruct((B, D), jnp.float32),
            jax.ShapeDtypeStruct((B,), jnp.float32),
        ),
        scratch_types=[
            pltpu.VMEM((NCH, C), jnp.int32),
            pltpu.VMEM((NCH, C), jnp.float32),
            pltpu.VMEM((C, D), jnp.float32),
            pltpu.VMEM((C, D), jnp.float32),
            pltpu.VMEM((C, D), jnp.float32),
            pltpu.VMEM((C, D), jnp.float32),
            pltpu.VMEM((C,), jnp.float32),
            pltpu.VMEM((C,), jnp.float32),
            pltpu.SemaphoreType.DMA,
            pltpu.SemaphoreType.DMA,
            pltpu.SemaphoreType.DMA,
            pltpu.SemaphoreType.DMA,
        ],
    )
    def k(nids_hbm, ts_hbm, left_hbm, right_hbm, pts_hbm,
          hl_out, hr_out, pts_out,
          idx_v, ts_v, lrows0, lrows1, rrows0, rrows1, pts0, pts1,
          gsem0, gsem1, wsem0, wsem1):
        wid = lax.axis_index("s") * NC + lax.axis_index("c")
        base = wid * b_per_w
        bufs = ((lrows0, rrows0, pts0), (lrows1, rrows1, pts1))
        gsems = (gsem0, gsem1)
        wsems = (wsem0, wsem1)
        pltpu.sync_copy(nids_hbm.at[wid], idx_v)
        pltpu.sync_copy(ts_hbm.at[wid], ts_v)

        def fire_gather(j, slot):
            ij = idx_v.at[j]
            l, r, p = bufs[slot]
            return (pltpu.async_copy(left_hbm.at[ij], l, gsems[slot]),
                    pltpu.async_copy(right_hbm.at[ij], r, gsems[slot]),
                    pltpu.async_copy(pts_hbm.at[ij], p, gsems[slot]))

        pending_g = [fire_gather(0, 0), None]
        pending_w = [None, None]
        for j in range(NCH):
            slot = j % 2
            nslot = 1 - slot
            if j + 1 < NCH:
                if pending_w[nslot] is not None:
                    for c in pending_w[nslot]:
                        c.wait()
                pending_g[nslot] = fire_gather(j + 1, nslot)
            for c in pending_g[slot]:
                c.wait()
            l, r, p = bufs[slot]
            for i in range(C // 16):
                sl = pl.ds(i * 16, 16)
                p[sl] = jnp.minimum(p[sl], ts_v[j, sl])
            o = pl.ds(base + j * C, C)
            pending_w[slot] = (
                pltpu.async_copy(l, hl_out.at[o], wsems[slot]),
                pltpu.async_copy(r, hr_out.at[o], wsems[slot]),
                pltpu.async_copy(p, pts_out.at[o], wsems[slot]))
        for pw in pending_w:
            if pw is not None:
                for c in pw:
                    c.wait()

    return k, NW, NCH, C


def kernel(nids, ts, left_weight, right_weight, prev_ts_table):
    B, = nids.shape
    N, D = left_weight.shape
    info = plsc.get_sparse_core_info()
    k, NW, NCH, C = _build(B, D, N, info.num_cores, info.num_subcores)
    nids3 = nids.astype(jnp.int32).reshape(NW, NCH, C)
    ts3 = ts.reshape(NW, NCH, C)
    h_left, h_right, prev_ts = k(nids3, ts3, left_weight, right_weight,
                                 prev_ts_table)
    return (h_left, h_right, prev_ts)


# speedup vs baseline: 1.7592x; 1.0460x over previous
import functools

import jax
import jax.numpy as jnp
from jax import lax
from jax.experimental import pallas as pl
from jax.experimental.pallas import tpu as pltpu
from jax.experimental.pallas import tpu_sc as plsc


@functools.lru_cache(maxsize=None)
def _build(B, D, N, NC, NS):
    NW = NC * NS
    b_per_w = B // NW
    C = 128
    NCH = b_per_w // C
    NSLOT = 3

    mesh = plsc.VectorSubcoreMesh(core_axis_name="c", subcore_axis_name="s")

    @functools.partial(
        pl.kernel,
        mesh=mesh,
        out_type=(
            jax.ShapeDtypeStruct((B, D), jnp.float32),
            jax.ShapeDtypeStruct((B, D), jnp.float32),
            jax.ShapeDtypeStruct((B,), jnp.float32),
        ),
        scratch_types=(
            [pltpu.VMEM((NCH, C), jnp.int32),
             pltpu.VMEM((NCH, C), jnp.float32)]
            + [pltpu.VMEM((C, D), jnp.float32) for _ in range(2 * NSLOT)]
            + [pltpu.VMEM((C,), jnp.float32) for _ in range(NSLOT)]
            + [pltpu.SemaphoreType.DMA for _ in range(2 * NSLOT)]
        ),
    )
    def k(nids_hbm, ts_hbm, left_hbm, right_hbm, pts_hbm,
          hl_out, hr_out, pts_out,
          idx_v, ts_v, *rest):
        rowbufs = rest[:2 * NSLOT]
        ptsbufs = rest[2 * NSLOT:3 * NSLOT]
        gsems = rest[3 * NSLOT:4 * NSLOT]
        wsems = rest[4 * NSLOT:5 * NSLOT]
        bufs = tuple((rowbufs[2 * s], rowbufs[2 * s + 1], ptsbufs[s])
                     for s in range(NSLOT))
        wid = lax.axis_index("s") * NC + lax.axis_index("c")
        base = wid * b_per_w
        pltpu.sync_copy(nids_hbm.at[wid], idx_v)
        tscp = pltpu.async_copy(ts_hbm.at[wid], ts_v, wsems[0])

        def fire_gather(j, slot):
            ij = idx_v.at[j]
            l, r, p = bufs[slot]
            return (pltpu.async_copy(left_hbm.at[ij], l, gsems[slot]),
                    pltpu.async_copy(right_hbm.at[ij], r, gsems[slot]),
                    pltpu.async_copy(pts_hbm.at[ij], p, gsems[slot]))

        AHEAD = NSLOT - 1
        pending_g = [None] * NSLOT
        pending_w = [None] * NSLOT
        for j0 in range(min(AHEAD, NCH)):
            pending_g[j0 % NSLOT] = fire_gather(j0, j0 % NSLOT)
        tscp.wait()
        for j in range(NCH):
            slot = j % NSLOT
            f = j + AHEAD
            if f < NCH:
                fslot = f % NSLOT
                if pending_w[fslot] is not None:
                    for c in pending_w[fslot]:
                        c.wait()
                    pending_w[fslot] = None
                pending_g[fslot] = fire_gather(f, fslot)
            for c in pending_g[slot]:
                c.wait()
            l, r, p = bufs[slot]
            for i in range(C // 16):
                sl = pl.ds(i * 16, 16)
                p[sl] = jnp.minimum(p[sl], ts_v[j, sl])
            o = pl.ds(base + j * C, C)
            pending_w[slot] = (
                pltpu.async_copy(l, hl_out.at[o], wsems[slot]),
                pltpu.async_copy(r, hr_out.at[o], wsems[slot]),
                pltpu.async_copy(p, pts_out.at[o], wsems[slot]))
        for pw in pending_w:
            if pw is not None:
                for c in pw:
                    c.wait()

    return k, NW, NCH, C


def kernel(nids, ts, left_weight, right_weight, prev_ts_table):
    B, = nids.shape
    N, D = left_weight.shape
    info = plsc.get_sparse_core_info()
    k, NW, NCH, C = _build(B, D, N, info.num_cores, info.num_subcores)
    nids3 = nids.astype(jnp.int32).reshape(NW, NCH, C)
    ts3 = ts.reshape(NW, NCH, C)
    h_left, h_right, prev_ts = k(nids3, ts3, left_weight, right_weight,
                                 prev_ts_table)
    return (h_left, h_right, prev_ts)


# speedup vs baseline: 1.7605x; 1.0007x over previous
import functools

import jax
import jax.numpy as jnp
from jax import lax
from jax.experimental import pallas as pl
from jax.experimental.pallas import tpu as pltpu
from jax.experimental.pallas import tpu_sc as plsc


@functools.lru_cache(maxsize=None)
def _build(B, D, N, NC, NS):
    NW = NC * NS
    b_per_w = B // NW
    C = 64
    NCH = b_per_w // C
    NSLOT = 6

    mesh = plsc.VectorSubcoreMesh(core_axis_name="c", subcore_axis_name="s")

    @functools.partial(
        pl.kernel,
        mesh=mesh,
        out_type=(
            jax.ShapeDtypeStruct((B, D), jnp.float32),
            jax.ShapeDtypeStruct((B, D), jnp.float32),
            jax.ShapeDtypeStruct((B,), jnp.float32),
        ),
        scratch_types=(
            [pltpu.VMEM((NCH, C), jnp.int32),
             pltpu.VMEM((NCH, C), jnp.float32)]
            + [pltpu.VMEM((C, D), jnp.float32) for _ in range(2 * NSLOT)]
            + [pltpu.VMEM((C,), jnp.float32) for _ in range(NSLOT)]
            + [pltpu.SemaphoreType.DMA for _ in range(2 * NSLOT)]
        ),
    )
    def k(nids_hbm, ts_hbm, left_hbm, right_hbm, pts_hbm,
          hl_out, hr_out, pts_out,
          idx_v, ts_v, *rest):
        rowbufs = rest[:2 * NSLOT]
        ptsbufs = rest[2 * NSLOT:3 * NSLOT]
        gsems = rest[3 * NSLOT:4 * NSLOT]
        wsems = rest[4 * NSLOT:5 * NSLOT]
        bufs = tuple((rowbufs[2 * s], rowbufs[2 * s + 1], ptsbufs[s])
                     for s in range(NSLOT))
        wid = lax.axis_index("s") * NC + lax.axis_index("c")
        base = wid * b_per_w
        pltpu.sync_copy(nids_hbm.at[wid], idx_v)
        tscp = pltpu.async_copy(ts_hbm.at[wid], ts_v, wsems[0])

        def fire_gather(j, slot):
            ij = idx_v.at[j]
            l, r, p = bufs[slot]
            return (pltpu.async_copy(left_hbm.at[ij], l, gsems[slot]),
                    pltpu.async_copy(right_hbm.at[ij], r, gsems[slot]),
                    pltpu.async_copy(pts_hbm.at[ij], p, gsems[slot]))

        AHEAD = NSLOT - 1
        pending_g = [None] * NSLOT
        pending_w = [None] * NSLOT
        for j0 in range(min(AHEAD, NCH)):
            pending_g[j0 % NSLOT] = fire_gather(j0, j0 % NSLOT)
        tscp.wait()
        for j in range(NCH):
            slot = j % NSLOT
            f = j + AHEAD
            if f < NCH:
                fslot = f % NSLOT
                if pending_w[fslot] is not None:
                    for c in pending_w[fslot]:
                        c.wait()
                    pending_w[fslot] = None
                pending_g[fslot] = fire_gather(f, fslot)
            for c in pending_g[slot]:
                c.wait()
            l, r, p = bufs[slot]
            for i in range(C // 16):
                sl = pl.ds(i * 16, 16)
                p[sl] = jnp.minimum(p[sl], ts_v[j, sl])
            o = pl.ds(base + j * C, C)
            pending_w[slot] = (
                pltpu.async_copy(l, hl_out.at[o], wsems[slot]),
                pltpu.async_copy(r, hr_out.at[o], wsems[slot]),
                pltpu.async_copy(p, pts_out.at[o], wsems[slot]))
        for pw in pending_w:
            if pw is not None:
                for c in pw:
                    c.wait()

    return k, NW, NCH, C


def kernel(nids, ts, left_weight, right_weight, prev_ts_table):
    B, = nids.shape
    N, D = left_weight.shape
    info = plsc.get_sparse_core_info()
    k, NW, NCH, C = _build(B, D, N, info.num_cores, info.num_subcores)
    nids3 = nids.astype(jnp.int32).reshape(NW, NCH, C)
    ts3 = ts.reshape(NW, NCH, C)
    h_left, h_right, prev_ts = k(nids3, ts3, left_weight, right_weight,
                                 prev_ts_table)
    return (h_left, h_right, prev_ts)
